# Initial kernel scaffold; baseline (speedup 1.0000x reference)
import jax
import jax.numpy as jnp
from jax.experimental import pallas as pl


def kernel(x, emb, W1, b1, W2, b2):
    raise NotImplementedError("write your pallas kernel here")


# speedup vs baseline: 68.0913x; 68.0913x over previous
import jax
import jax.numpy as jnp
from jax import lax
from jax.experimental import pallas as pl
from jax.experimental.pallas import tpu as pltpu
from jax.experimental.pallas import tpu_sc as plsc

B = 16384
L = 200
D = 8
NW = 32
ROWS_PER_W = B // NW
CB = 16
NCH = ROWS_PER_W // CB
CH_IDX = CB * L


def _pool_body(x_hbm, emb_hbm, out_hbm, idx_v, rows_v, pooled_v, sem):
    wid = lax.axis_index("c") * 16 + lax.axis_index("s")
    base_idx = wid * (ROWS_PER_W * L)
    inv_l = jnp.float32(1.0 / L)

    l16 = lax.iota(jnp.int32, 16)
    col = lax.bitwise_and(l16, 7)
    half = lax.shift_right_logical(l16, 3) * L

    def chunk_body(g, carry):
        pltpu.sync_copy(x_hbm.at[pl.ds(base_idx + g * CH_IDX, CH_IDX)], idx_v)
        pltpu.async_copy(emb_hbm.at[idx_v], rows_v, sem).wait()
        for p in range(CB // 2):
            row0 = (2 * p) * L + half
            def jbody(j, acc):
                return acc + plsc.load_gather(rows_v, [row0 + j, col])
            acc = lax.fori_loop(0, L, jbody, jnp.zeros((16,), jnp.float32))
            pooled_v[pl.ds(g * (CB * D) + p * 16, 16)] = acc * inv_l
        return carry

    lax.fori_loop(0, NCH, chunk_body, 0)
    pltpu.sync_copy(pooled_v, out_hbm.at[pl.ds(wid * (ROWS_PER_W * D),
                                               ROWS_PER_W * D)])


_pool = pl.kernel(
    _pool_body,
    out_type=jax.ShapeDtypeStruct((B * D,), jnp.float32),
    mesh=plsc.VectorSubcoreMesh(core_axis_name="c", subcore_axis_name="s"),
    compiler_params=pltpu.CompilerParams(needs_layout_passes=False,
                                         use_tc_tiling_on_sc=False),
    scratch_types=[
        pltpu.VMEM((CH_IDX,), jnp.int32),
        pltpu.VMEM((CH_IDX, D), jnp.float32),
        pltpu.VMEM((ROWS_PER_W * D,), jnp.float32),
        pltpu.SemaphoreType.DMA,
    ],
)


def _mlp_body(h_ref, w1_ref, b1_ref, w2_ref, b2_ref, out_ref):
    h = h_ref[...]
    a = jnp.dot(h, w1_ref[...], preferred_element_type=jnp.float32)
    a = jnp.maximum(a + b1_ref[...], 0.0)
    z = jnp.sum(a * w2_ref[...][:, 0][None, :], axis=1, keepdims=True)
    z = z + b2_ref[...]
    out_ref[...] = 1.0 / (1.0 + jnp.exp(-z))


def _mlp(pooled, w1, b1, w2, b2):
    return pl.pallas_call(
        _mlp_body,
        out_shape=jax.ShapeDtypeStruct((B, 1), jnp.float32),
    )(pooled, w1, b1, w2, b2)


@jax.jit
def kernel(x, emb, W1, b1, W2, b2):
    x_flat = x.reshape(-1).astype(jnp.int32)
    pooled = _pool(x_flat, emb).reshape(B, D)
    return _mlp(pooled, W1, b1, W2, b2)


# speedup vs baseline: 80.3201x; 1.1796x over previous
import jax
import jax.numpy as jnp
from jax import lax
from jax.experimental import pallas as pl
from jax.experimental.pallas import tpu as pltpu
from jax.experimental.pallas import tpu_sc as plsc

B = 16384
L = 200
D = 8
NW = 32
ROWS_PER_W = B // NW
CB = 16
NCH = ROWS_PER_W // CB
CH_IDX = CB * L


def _pool_body(x_hbm, emb_hbm, out_hbm, idx_v, rows_v, pooled_v, sem):
    wid = lax.axis_index("c") * 16 + lax.axis_index("s")
    base_idx = wid * (ROWS_PER_W * L)
    inv_l = jnp.float32(1.0 / L)

    l16 = lax.iota(jnp.int32, 16)
    col = lax.bitwise_and(l16, 7)
    half = lax.shift_right_logical(l16, 3) * L

    def chunk_body(g, carry):
        pltpu.sync_copy(x_hbm.at[pl.ds(base_idx + g * CH_IDX, CH_IDX)], idx_v)
        pltpu.async_copy(emb_hbm.at[idx_v], rows_v, sem).wait()

        def jbody(j, accs):
            rj = half + j
            return tuple(
                acc + plsc.load_gather(rows_v, [(2 * p) * L + rj, col])
                for p, acc in enumerate(accs))

        accs = lax.fori_loop(
            0, L, jbody,
            tuple(jnp.zeros((16,), jnp.float32) for _ in range(CB // 2)))
        for p, acc in enumerate(accs):
            pooled_v[pl.ds(g * (CB * D) + p * 16, 16)] = acc * inv_l
        return carry

    lax.fori_loop(0, NCH, chunk_body, 0)
    pltpu.sync_copy(pooled_v, out_hbm.at[pl.ds(wid * (ROWS_PER_W * D),
                                               ROWS_PER_W * D)])


_pool = pl.kernel(
    _pool_body,
    out_type=jax.ShapeDtypeStruct((B * D,), jnp.float32),
    mesh=plsc.VectorSubcoreMesh(core_axis_name="c", subcore_axis_name="s"),
    compiler_params=pltpu.CompilerParams(needs_layout_passes=False,
                                         use_tc_tiling_on_sc=False),
    scratch_types=[
        pltpu.VMEM((CH_IDX,), jnp.int32),
        pltpu.VMEM((CH_IDX, D), jnp.float32),
        pltpu.VMEM((ROWS_PER_W * D,), jnp.float32),
        pltpu.SemaphoreType.DMA,
    ],
)


def _mlp_body(h_ref, w1_ref, b1_ref, w2_ref, b2_ref, out_ref):
    h = h_ref[...]
    a = jnp.dot(h, w1_ref[...], preferred_element_type=jnp.float32)
    a = jnp.maximum(a + b1_ref[...], 0.0)
    z = jnp.sum(a * w2_ref[...][:, 0][None, :], axis=1, keepdims=True)
    z = z + b2_ref[...]
    out_ref[...] = 1.0 / (1.0 + jnp.exp(-z))


def _mlp(pooled, w1, b1, w2, b2):
    return pl.pallas_call(
        _mlp_body,
        out_shape=jax.ShapeDtypeStruct((B, 1), jnp.float32),
    )(pooled, w1, b1, w2, b2)


@jax.jit
def kernel(x, emb, W1, b1, W2, b2):
    x_flat = x.reshape(-1).astype(jnp.int32)
    pooled = _pool(x_flat, emb).reshape(B, D)
    return _mlp(pooled, W1, b1, W2, b2)
